# baseline (speedup 1.0000x reference)
import functools

import jax
import jax.numpy as jnp
from jax import lax
from jax.experimental import pallas as pl
from jax.experimental.pallas import tpu as pltpu
from jax.experimental.pallas import tpu_sc as plsc

D = 64
T = 200
B = 4096
NC, NS = 2, 16
NW = NC * NS
ROWS_PER_W = B // NW
HALF = T // 2
NBUF = 2

_mesh = plsc.VectorSubcoreMesh(core_axis_name="c", subcore_axis_name="s")


@functools.partial(
    pl.kernel,
    out_type=jax.ShapeDtypeStruct((B, T, D), jnp.float32),
    mesh=_mesh,
    scratch_types=[
        pltpu.VMEM((ROWS_PER_W, 2, HALF), jnp.int32),
        [pltpu.VMEM((T, D), jnp.float32)] * NBUF,
        [pltpu.VMEM((T, D), jnp.float32)] * NBUF,
        pltpu.VMEM((T, D), jnp.float32),
        [pltpu.SemaphoreType.DMA] * NBUF,
        [pltpu.SemaphoreType.DMA] * NBUF,
    ],
    compiler_params=pltpu.CompilerParams(use_tc_tiling_on_sc=False),
)
def _embed(x_hbm, tok_hbm, pos_hbm, out_hbm,
           idx_v, gbuf, sbuf, pos_v, gsem, ssem):
    wid = lax.axis_index("s") * NC + lax.axis_index("c")
    pltpu.sync_copy(pos_hbm, pos_v)
    pltpu.sync_copy(x_hbm.at[wid], idx_v)

    def start_gather(i, nb):
        pltpu.async_copy(
            tok_hbm.at[idx_v.at[i, 0]], gbuf[nb].at[pl.ds(0, HALF)], gsem[nb])
        pltpu.async_copy(
            tok_hbm.at[idx_v.at[i, 1]], gbuf[nb].at[pl.ds(HALF, HALF)],
            gsem[nb])

    for nb in range(NBUF):
        start_gather(nb, nb)

    @pl.loop(0, ROWS_PER_W, step=NBUF)
    def _(i0):
        for nb in range(NBUF):
            i = i0 + nb
            pltpu.make_async_copy(
                tok_hbm.at[pl.ds(0, T)], gbuf[nb], gsem[nb]).wait()
            @pl.when(i >= NBUF)
            def _():
                pltpu.make_async_copy(
                    tok_hbm.at[pl.ds(0, T)], sbuf[nb], ssem[nb]).wait()

            @pl.loop(0, T, unroll=2)
            def _(r):
                for c in range(D // 16):
                    sl = pl.ds(c * 16, 16)
                    sbuf[nb][r, sl] = gbuf[nb][r, sl] + pos_v[r, sl]

            @pl.when(i + NBUF < ROWS_PER_W)
            def _():
                start_gather(i + NBUF, nb)
            pltpu.async_copy(
                sbuf[nb], out_hbm.at[wid * ROWS_PER_W + i], ssem[nb])

    for nb in range(NBUF):
        pltpu.make_async_copy(
            tok_hbm.at[pl.ds(0, T)], sbuf[nb], ssem[nb]).wait()


def kernel(x, token_table, pos_table):
    x32 = x.astype(jnp.int32).reshape(NW, ROWS_PER_W, 2, HALF)
    return _embed(x32, token_table, pos_table)
